# baseline (speedup 1.0000x reference)
import functools

import jax
import jax.numpy as jnp
from jax import lax
from jax.experimental import pallas as pl
from jax.experimental.pallas import tpu as pltpu
from jax.experimental.pallas import tpu_sc as plsc

B, D_IN, H, NZ, NC = 4096, 768, 512, 64, 8192
BLK = 256
GRID = B // BLK

_SC_INFO = plsc.get_sparse_core_info()
_NCORES = _SC_INFO.num_cores
_NSUB = _SC_INFO.num_subcores
_NW = _NCORES * _NSUB
_BINS_PER_W = NC // _NW
_LANES = 16


def _tc_body(inp_ref, w1_ref, b1_ref, w2_ref, b2_ref, cent_ref,
             loss_ref, assign_ref, soft_ref, csq_ref):
    i = pl.program_id(0)

    cent = cent_ref[...]

    @pl.when(i == 0)
    def _precompute():
        csq_ref[...] = jnp.broadcast_to(
            jnp.sum(cent * cent, axis=1)[None, :], (8, NC))

    x = inp_ref[...]
    h = jnp.maximum(
        jax.lax.dot_general(x, w1_ref[...], (((1,), (0,)), ((), ())),
                            preferred_element_type=jnp.float32)
        + b1_ref[...], 0.0)
    fv = jax.lax.dot_general(h, w2_ref[...], (((1,), (0,)), ((), ())),
                             preferred_element_type=jnp.float32) \
        + b2_ref[...]

    neg2dots = jax.lax.dot_general(fv * -2.0, cent, (((1,), (1,)), ((), ())),
                                   preferred_element_type=jnp.float32)
    f_sq = jnp.sum(fv * fv, axis=1, keepdims=True)
    c_sq = csq_ref[...]
    d2 = jnp.maximum(
        ((f_sq.reshape(BLK // 8, 8, 1) + c_sq[None, :, :])
         + neg2dots.reshape(BLK // 8, 8, NC)).reshape(BLK, NC), 0.0)
    d = jnp.sqrt(d2 + 1e-12)

    min_d = jnp.min(d, axis=1)
    cols = jax.lax.broadcasted_iota(jnp.int32, (BLK, NC), 1)
    am = jnp.min(jnp.where(d == min_d[:, None], cols, NC), axis=1)
    assign_ref[...] = am.astype(jnp.int32)

    p = jnp.exp(min_d[:, None] - d)
    z = jnp.sum(p, axis=1)
    soft_add = jax.lax.dot_general((1.0 / z)[None, :], p,
                                   (((1,), (0,)), ((), ())),
                                   preferred_element_type=jnp.float32)[0]

    @pl.when(i == 0)
    def _init():
        loss_ref[...] = jnp.zeros_like(loss_ref)
        soft_ref[...] = jnp.zeros_like(soft_ref)

    loss_ref[...] += jnp.sum(min_d).reshape(1, 1)
    soft_ref[...] += soft_add


def _sc_counts_body(assign_hbm, out_hbm, assign_v, counts_v):
    wid = lax.axis_index("s") * _NCORES + lax.axis_index("c")
    base = wid * _BINS_PER_W

    pltpu.sync_copy(assign_hbm, assign_v)

    zeros = jnp.zeros((_LANES,), jnp.int32)
    for j in range(_BINS_PER_W // _LANES):
        counts_v[pl.ds(j * _LANES, _LANES)] = zeros

    ones = jnp.ones((_LANES,), jnp.int32)

    def step(k, carry):
        idx = assign_v[pl.ds(k * _LANES, _LANES)] - base
        mask = (idx >= 0) & (idx < _BINS_PER_W)
        plsc.addupdate_scatter(counts_v, [idx], ones, mask=mask)
        return carry

    lax.fori_loop(0, B // _LANES, step, 0)

    pltpu.sync_copy(counts_v, out_hbm.at[pl.ds(base, _BINS_PER_W)])


@functools.partial(
    pl.kernel,
    mesh=plsc.VectorSubcoreMesh(core_axis_name="c", subcore_axis_name="s"),
    out_type=jax.ShapeDtypeStruct((NC,), jnp.int32),
    scratch_types=[
        pltpu.VMEM((B,), jnp.int32),
        pltpu.VMEM((_BINS_PER_W,), jnp.int32),
    ],
    compiler_params=pltpu.CompilerParams(needs_layout_passes=False),
)
def _sc_counts(assign_hbm, out_hbm, assign_v, counts_v):
    _sc_counts_body(assign_hbm, out_hbm, assign_v, counts_v)


def kernel(inp, W1, b1, W2, b2, centroids):
    loss_sum, assigns, soft_counts = pl.pallas_call(
        _tc_body,
        grid=(GRID,),
        in_specs=[
            pl.BlockSpec((BLK, D_IN), lambda i: (i, 0)),
            pl.BlockSpec((D_IN, H), lambda i: (0, 0)),
            pl.BlockSpec((1, H), lambda i: (0, 0)),
            pl.BlockSpec((H, NZ), lambda i: (0, 0)),
            pl.BlockSpec((1, NZ), lambda i: (0, 0)),
            pl.BlockSpec((NC, NZ), lambda i: (0, 0)),
        ],
        out_specs=[
            pl.BlockSpec((1, 1), lambda i: (0, 0)),
            pl.BlockSpec((BLK,), lambda i: (i,)),
            pl.BlockSpec((NC,), lambda i: (0,)),
        ],
        out_shape=[
            jax.ShapeDtypeStruct((1, 1), jnp.float32),
            jax.ShapeDtypeStruct((B,), jnp.int32),
            jax.ShapeDtypeStruct((NC,), jnp.float32),
        ],
        scratch_shapes=[pltpu.VMEM((8, NC), jnp.float32)],
    )(inp, W1, b1[None, :], W2, b2[None, :], centroids)
    raw_counts = _sc_counts(assigns)
    cluster_loss = loss_sum[0, 0] / B
    return (cluster_loss, assigns, soft_counts, raw_counts)


# speedup vs baseline: 1.0638x; 1.0638x over previous
import functools

import jax
import jax.numpy as jnp
from jax import lax
from jax.experimental import pallas as pl
from jax.experimental.pallas import tpu as pltpu
from jax.experimental.pallas import tpu_sc as plsc

B, D_IN, H, NZ, NC = 4096, 768, 512, 64, 8192
BLK = 256
GRID = B // BLK

_SC_INFO = plsc.get_sparse_core_info()
_NCORES = _SC_INFO.num_cores
_NSUB = _SC_INFO.num_subcores
_NW = _NCORES * _NSUB
_BINS_PER_W = NC // _NW
_LANES = 16


def _tc_body(inp_ref, w1_ref, b1_ref, w2_ref, b2_ref, cent_ref,
             loss_ref, assign_ref, soft_ref, csq_ref):
    i = pl.program_id(0)

    cent = cent_ref[...]

    @pl.when(i == 0)
    def _precompute():
        csq_ref[...] = jnp.broadcast_to(
            jnp.sum(cent * cent, axis=1)[None, :], (8, NC))

    x = inp_ref[...]
    h = jnp.maximum(
        jax.lax.dot_general(x, w1_ref[...], (((1,), (0,)), ((), ())),
                            preferred_element_type=jnp.float32)
        + b1_ref[...], 0.0)
    fv = jax.lax.dot_general(h, w2_ref[...], (((1,), (0,)), ((), ())),
                             preferred_element_type=jnp.float32) \
        + b2_ref[...]

    neg2dots = jax.lax.dot_general(fv * -2.0, cent, (((1,), (1,)), ((), ())),
                                   preferred_element_type=jnp.float32)
    f_sq = jnp.sum(fv * fv, axis=1, keepdims=True)
    c_sq = csq_ref[...]
    d2 = jnp.maximum(
        ((f_sq.reshape(BLK // 8, 8, 1) + c_sq[None, :, :])
         + neg2dots.reshape(BLK // 8, 8, NC)).reshape(BLK, NC), 0.0)

    min_d2 = jnp.min(d2, axis=1)
    cols = jax.lax.broadcasted_iota(jnp.int32, (BLK, NC), 1)
    am = jnp.min(jnp.where(d2 == min_d2[:, None], cols, NC), axis=1)
    assign_ref[...] = am.astype(jnp.int32)
    min_d = jnp.sqrt(min_d2 + 1e-12)

    y = d2 + 1e-12
    d = y * jax.lax.rsqrt(y)

    p = jnp.exp(min_d[:, None] - d)
    z = jnp.sum(p, axis=1)
    soft_add = jax.lax.dot_general((1.0 / z)[None, :], p,
                                   (((1,), (0,)), ((), ())),
                                   preferred_element_type=jnp.float32)[0]

    @pl.when(i == 0)
    def _init():
        loss_ref[...] = jnp.zeros_like(loss_ref)
        soft_ref[...] = jnp.zeros_like(soft_ref)

    loss_ref[...] += jnp.sum(min_d).reshape(1, 1)
    soft_ref[...] += soft_add


def _sc_counts_body(assign_hbm, out_hbm, assign_v, counts_v):
    wid = lax.axis_index("s") * _NCORES + lax.axis_index("c")
    base = wid * _BINS_PER_W

    pltpu.sync_copy(assign_hbm, assign_v)

    zeros = jnp.zeros((_LANES,), jnp.int32)
    for j in range(_BINS_PER_W // _LANES):
        counts_v[pl.ds(j * _LANES, _LANES)] = zeros

    ones = jnp.ones((_LANES,), jnp.int32)

    def step(k, carry):
        idx = assign_v[pl.ds(k * _LANES, _LANES)] - base
        mask = (idx >= 0) & (idx < _BINS_PER_W)
        plsc.addupdate_scatter(counts_v, [idx], ones, mask=mask)
        return carry

    lax.fori_loop(0, B // _LANES, step, 0)

    pltpu.sync_copy(counts_v, out_hbm.at[pl.ds(base, _BINS_PER_W)])


@functools.partial(
    pl.kernel,
    mesh=plsc.VectorSubcoreMesh(core_axis_name="c", subcore_axis_name="s"),
    out_type=jax.ShapeDtypeStruct((NC,), jnp.int32),
    scratch_types=[
        pltpu.VMEM((B,), jnp.int32),
        pltpu.VMEM((_BINS_PER_W,), jnp.int32),
    ],
    compiler_params=pltpu.CompilerParams(needs_layout_passes=False),
)
def _sc_counts(assign_hbm, out_hbm, assign_v, counts_v):
    _sc_counts_body(assign_hbm, out_hbm, assign_v, counts_v)


def kernel(inp, W1, b1, W2, b2, centroids):
    loss_sum, assigns, soft_counts = pl.pallas_call(
        _tc_body,
        grid=(GRID,),
        in_specs=[
            pl.BlockSpec((BLK, D_IN), lambda i: (i, 0)),
            pl.BlockSpec((D_IN, H), lambda i: (0, 0)),
            pl.BlockSpec((1, H), lambda i: (0, 0)),
            pl.BlockSpec((H, NZ), lambda i: (0, 0)),
            pl.BlockSpec((1, NZ), lambda i: (0, 0)),
            pl.BlockSpec((NC, NZ), lambda i: (0, 0)),
        ],
        out_specs=[
            pl.BlockSpec((1, 1), lambda i: (0, 0)),
            pl.BlockSpec((BLK,), lambda i: (i,)),
            pl.BlockSpec((NC,), lambda i: (0,)),
        ],
        out_shape=[
            jax.ShapeDtypeStruct((1, 1), jnp.float32),
            jax.ShapeDtypeStruct((B,), jnp.int32),
            jax.ShapeDtypeStruct((NC,), jnp.float32),
        ],
        scratch_shapes=[pltpu.VMEM((8, NC), jnp.float32)],
    )(inp, W1, b1[None, :], W2, b2[None, :], centroids)
    raw_counts = _sc_counts(assigns)
    cluster_loss = loss_sum[0, 0] / B
    return (cluster_loss, assigns, soft_counts, raw_counts)


# speedup vs baseline: 1.2403x; 1.1659x over previous
import functools

import jax
import jax.numpy as jnp
from jax import lax
from jax.experimental import pallas as pl
from jax.experimental.pallas import tpu as pltpu
from jax.experimental.pallas import tpu_sc as plsc

B, D_IN, H, NZ, NC = 4096, 768, 512, 64, 8192
BLK = 512
GRID = B // BLK

_SC_INFO = plsc.get_sparse_core_info()
_NCORES = _SC_INFO.num_cores
_NSUB = _SC_INFO.num_subcores
_NW = _NCORES * _NSUB
_BINS_PER_W = NC // _NW
_LANES = 16


def _tc_body(inp_ref, w1_ref, b1_ref, w2_ref, b2_ref, cent_ref,
             loss_ref, assign_ref, soft_ref, csq_ref):
    i = pl.program_id(0)

    cent = cent_ref[...]

    @pl.when(i == 0)
    def _precompute():
        csq_ref[...] = jnp.broadcast_to(
            jnp.sum(cent * cent, axis=1)[None, :], (8, NC))

    x = inp_ref[...]
    h = jnp.maximum(
        jax.lax.dot_general(x, w1_ref[...], (((1,), (0,)), ((), ())),
                            preferred_element_type=jnp.float32)
        + b1_ref[...], 0.0)
    fv = jax.lax.dot_general(h, w2_ref[...], (((1,), (0,)), ((), ())),
                             preferred_element_type=jnp.float32) \
        + b2_ref[...]

    neg2dots = jax.lax.dot_general(fv * -2.0, cent, (((1,), (1,)), ((), ())),
                                   preferred_element_type=jnp.float32)
    f_sq = jnp.sum(fv * fv, axis=1, keepdims=True)
    c_sq = csq_ref[...]
    d2 = jnp.maximum(
        ((f_sq.reshape(BLK // 8, 8, 1) + c_sq[None, :, :])
         + neg2dots.reshape(BLK // 8, 8, NC)).reshape(BLK, NC), 0.0)

    min_d2 = jnp.min(d2, axis=1)
    cols = jax.lax.broadcasted_iota(jnp.int32, (BLK, NC), 1)
    am = jnp.min(jnp.where(d2 == min_d2[:, None], cols, NC), axis=1)
    assign_ref[...] = am.astype(jnp.int32)
    min_d = jnp.sqrt(min_d2 + 1e-12)

    y = d2 + 1e-12
    d = y * jax.lax.rsqrt(y)

    p = jnp.exp(min_d[:, None] - d)
    z = jnp.sum(p, axis=1)
    soft_add = jax.lax.dot_general((1.0 / z)[None, :], p,
                                   (((1,), (0,)), ((), ())),
                                   preferred_element_type=jnp.float32)[0]

    @pl.when(i == 0)
    def _init():
        loss_ref[...] = jnp.zeros_like(loss_ref)
        soft_ref[...] = jnp.zeros_like(soft_ref)

    loss_ref[...] += jnp.sum(min_d).reshape(1, 1)
    soft_ref[...] += soft_add


def _sc_counts_body(assign_hbm, out_hbm, assign_v, counts_v):
    wid = lax.axis_index("s") * _NCORES + lax.axis_index("c")
    base = wid * _BINS_PER_W

    pltpu.sync_copy(assign_hbm, assign_v)

    zeros = jnp.zeros((_LANES,), jnp.int32)
    for j in range(_BINS_PER_W // _LANES):
        counts_v[pl.ds(j * _LANES, _LANES)] = zeros

    ones = jnp.ones((_LANES,), jnp.int32)

    def step(k, carry):
        idx = assign_v[pl.ds(k * _LANES, _LANES)] - base
        mask = (idx >= 0) & (idx < _BINS_PER_W)
        plsc.addupdate_scatter(counts_v, [idx], ones, mask=mask)
        return carry

    lax.fori_loop(0, B // _LANES, step, 0)

    pltpu.sync_copy(counts_v, out_hbm.at[pl.ds(base, _BINS_PER_W)])


@functools.partial(
    pl.kernel,
    mesh=plsc.VectorSubcoreMesh(core_axis_name="c", subcore_axis_name="s"),
    out_type=jax.ShapeDtypeStruct((NC,), jnp.int32),
    scratch_types=[
        pltpu.VMEM((B,), jnp.int32),
        pltpu.VMEM((_BINS_PER_W,), jnp.int32),
    ],
    compiler_params=pltpu.CompilerParams(needs_layout_passes=False),
)
def _sc_counts(assign_hbm, out_hbm, assign_v, counts_v):
    _sc_counts_body(assign_hbm, out_hbm, assign_v, counts_v)


def kernel(inp, W1, b1, W2, b2, centroids):
    loss_sum, assigns, soft_counts = pl.pallas_call(
        _tc_body,
        grid=(GRID,),
        in_specs=[
            pl.BlockSpec((BLK, D_IN), lambda i: (i, 0)),
            pl.BlockSpec((D_IN, H), lambda i: (0, 0)),
            pl.BlockSpec((1, H), lambda i: (0, 0)),
            pl.BlockSpec((H, NZ), lambda i: (0, 0)),
            pl.BlockSpec((1, NZ), lambda i: (0, 0)),
            pl.BlockSpec((NC, NZ), lambda i: (0, 0)),
        ],
        out_specs=[
            pl.BlockSpec((1, 1), lambda i: (0, 0)),
            pl.BlockSpec((BLK,), lambda i: (i,)),
            pl.BlockSpec((NC,), lambda i: (0,)),
        ],
        out_shape=[
            jax.ShapeDtypeStruct((1, 1), jnp.float32),
            jax.ShapeDtypeStruct((B,), jnp.int32),
            jax.ShapeDtypeStruct((NC,), jnp.float32),
        ],
        scratch_shapes=[pltpu.VMEM((8, NC), jnp.float32)],
    )(inp, W1, b1[None, :], W2, b2[None, :], centroids)
    raw_counts = _sc_counts(assigns)
    cluster_loss = loss_sum[0, 0] / B
    return (cluster_loss, assigns, soft_counts, raw_counts)
